# Initial kernel scaffold; baseline (speedup 1.0000x reference)
import jax
import jax.numpy as jnp
from jax.experimental import pallas as pl


def kernel(feat_a, feat_b, feat_c, W_a, W_b, W_c):
    raise NotImplementedError("write your pallas kernel here")


# speedup vs baseline: 4.9650x; 4.9650x over previous
import functools

import jax
import jax.numpy as jnp
from jax import lax
from jax.experimental import pallas as pl
from jax.experimental.pallas import tpu as pltpu
from jax.experimental.pallas import tpu_sc as plsc

_B, _L, _V, _D = 4096, 50, 100000, 32
_NW = 32
_CH = 128
_R = (_B * _L) // _NW
_NCH = _R // _CH


def _sc_body(ia, ib, ic, wa, wb, wc, out, idx_v, rows_v, gsem):
    wid = lax.axis_index("c") * 16 + lax.axis_index("s")
    base = wid * _R

    def one_table(t, idx_hbm, w_hbm):
        pltpu.sync_copy(idx_hbm.at[wid], idx_v)
        pltpu.async_copy(w_hbm.at[idx_v.at[0]], rows_v.at[0], gsem)

        def body(j, carry):
            pltpu.make_async_copy(w_hbm.at[pl.ds(0, _CH)], rows_v.at[0],
                                  gsem).wait()

            @pl.when(j + 1 < _NCH)
            def _():
                pltpu.async_copy(w_hbm.at[idx_v.at[j + 1]],
                                 rows_v.at[(j + 1) % 2], gsem)

            pltpu.sync_copy(
                rows_v.at[j % 2],
                out.at[pl.ds(base + j * _CH, _CH), pl.ds(t * _D, _D)])
            return carry

        lax.fori_loop(0, _NCH, body, 0)

    one_table(0, ia, wa)
    one_table(1, ib, wb)
    one_table(2, ic, wc)


_mesh = plsc.VectorSubcoreMesh(core_axis_name="c", subcore_axis_name="s")

_gather3 = functools.partial(
    pl.kernel,
    out_type=jax.ShapeDtypeStruct((_B * _L, 3 * _D), jnp.float32),
    mesh=_mesh,
    scratch_types=[
        pltpu.VMEM((_NCH, _CH), jnp.int32),
        pltpu.VMEM((2, _CH, _D), jnp.float32),
        pltpu.SemaphoreType.DMA,
    ],
    compiler_params=pltpu.CompilerParams(use_tc_tiling_on_sc=False),
)(_sc_body)


def kernel(feat_a, feat_b, feat_c, W_a, W_b, W_c):
    ia = feat_a.T.reshape(_NW, _NCH, _CH)
    ib = feat_b.T.reshape(_NW, _NCH, _CH)
    ic = feat_c.T.reshape(_NW, _NCH, _CH)
    out = _gather3(ia, ib, ic, W_a, W_b, W_c)
    return out.reshape(_L, _B, 3 * _D)
